# baseline (speedup 1.0000x reference)
import functools

import jax
import jax.numpy as jnp
from jax.experimental import pallas as pl
from jax.experimental.pallas import tpu as pltpu

_TILE = 256
_N_EXPERTS = 64
_K = 8


def _body(x_ref, wi_ref, bi_ref, g1_ref, be1_ref, wh1_ref, bh1_ref,
          g2_ref, be2_ref, wh2_ref, bh2_ref, wo_ref, bo_ref,
          tp_ref, ti_ref, gp_ref, lg_ref):
    logits = lg_ref[...]
    m = jnp.max(logits, axis=-1, keepdims=True)
    e = jnp.exp(logits - m)
    probs = e / jnp.sum(e, axis=-1, keepdims=True)
    gp_ref[...] = probs

    lane = jax.lax.broadcasted_iota(jnp.int32, probs.shape, 1)
    desc = (63 - lane).astype(jnp.float32)
    p = probs
    vals = []
    scs = []
    for _ in range(_K):
        mx = jnp.max(p, axis=-1, keepdims=True)
        score = jnp.where(p >= mx, desc, -1.0)
        sc = jnp.max(score, axis=-1, keepdims=True)
        vals.append(mx)
        scs.append(sc)
        p = jnp.where(score == sc, -1.0, p)
    tv = jnp.concatenate(vals, axis=1)
    tp_ref[...] = tv / jnp.sum(tv, axis=1, keepdims=True)
    ti_ref[...] = (63.0 - jnp.concatenate(scs, axis=1)).astype(jnp.int32)

    x = x_ref[...]
    h0 = jnp.dot(x, wi_ref[...], preferred_element_type=jnp.float32)
    h0 = jnp.maximum(h0 + bi_ref[...], 0.0)

    mu = jnp.mean(h0, axis=-1, keepdims=True)
    var = jnp.mean((h0 - mu) ** 2, axis=-1, keepdims=True)
    t = (h0 - mu) * jax.lax.rsqrt(var + 1e-5) * g1_ref[...] + be1_ref[...]
    t = jnp.maximum(t, 0.0)

    h1 = jnp.dot(t, wh1_ref[...], preferred_element_type=jnp.float32)
    h1 = h1 + bh1_ref[...] + h0

    mu2 = jnp.mean(h1, axis=-1, keepdims=True)
    var2 = jnp.mean((h1 - mu2) ** 2, axis=-1, keepdims=True)
    t2 = (h1 - mu2) * jax.lax.rsqrt(var2 + 1e-5) * g2_ref[...] + be2_ref[...]
    t2 = jnp.maximum(t2, 0.0)

    h2 = jnp.dot(t2, wh2_ref[...], preferred_element_type=jnp.float32)
    h2 = h2 + bh2_ref[...]

    new_logits = jnp.dot(h2, wo_ref[...], preferred_element_type=jnp.float32)
    lg_ref[...] = new_logits + bo_ref[...]


@functools.partial(jax.jit, static_argnames=("interpret",))
def _run(x, wi, bi, g1, be1, wh1, bh1, g2, be2, wh2, bh2, wo, bo,
         interpret=False):
    n = x.shape[0]
    d = x.shape[1]
    nblk = n // _TILE
    half = nblk // 2
    grid = (2, half + 1)

    def xmap(c, j):
        return (c * half + jnp.minimum(j, half - 1), 0)

    def omap(c, j):
        return (c * half + jnp.maximum(j - 1, 0), 0)

    def rep(c, j):
        return (0, 0)

    full = lambda s: pl.BlockSpec(s, rep)
    out_shapes = (
        jax.ShapeDtypeStruct((n, _K), jnp.float32),
        jax.ShapeDtypeStruct((n, _K), jnp.int32),
        jax.ShapeDtypeStruct((n, _N_EXPERTS), jnp.float32),
    )
    return pl.pallas_call(
        _body,
        grid=grid,
        in_specs=[
            pl.BlockSpec((_TILE, d), xmap),
            full((d, 256)), full((1, 256)), full((1, 256)), full((1, 256)),
            full((256, 256)), full((1, 256)), full((1, 256)), full((1, 256)),
            full((256, 128)), full((1, 128)),
            full((128, _N_EXPERTS)), full((1, _N_EXPERTS)),
        ],
        out_specs=(
            pl.BlockSpec((_TILE, _K), omap),
            pl.BlockSpec((_TILE, _K), omap),
            pl.BlockSpec((_TILE, _N_EXPERTS), omap),
        ),
        out_shape=out_shapes,
        scratch_shapes=[pltpu.VMEM((_TILE, _N_EXPERTS), jnp.float32)],
        compiler_params=pltpu.CompilerParams(
            dimension_semantics=("parallel", "arbitrary"),
        ),
        interpret=interpret,
    )(x, wi, bi, g1, be1, wh1, bh1, g2, be2, wh2, bh2, wo, bo)


def kernel(x, W_in, b_in, ln1_g, ln1_b, W_h1, b_h1, ln2_g, ln2_b,
           W_h2, b_h2, W_out, b_out, temperature):
    temp = jnp.clip(temperature, 0.5, 5.0)[0]
    wo = (W_out.T / temp).astype(jnp.float32)
    bo = (b_out / temp).reshape(1, -1)
    r = lambda v: v.reshape(1, -1)
    return _run(x, W_in.T, r(b_in), r(ln1_g), r(ln1_b), W_h1.T, r(b_h1),
                r(ln2_g), r(ln2_b), W_h2.T, r(b_h2), wo, bo)


# speedup vs baseline: 1.2861x; 1.2861x over previous
import functools

import jax
import jax.numpy as jnp
from jax.experimental import pallas as pl
from jax.experimental.pallas import tpu as pltpu

_TILE = 512
_N_EXPERTS = 64
_K = 8


def _body(x_ref, wi_ref, bi_ref, g1_ref, be1_ref, wh1_ref, bh1_ref,
          g2_ref, be2_ref, wh2_ref, bh2_ref, wo_ref, bo_ref,
          tp_ref, ti_ref, gp_ref, lg_ref):
    logits = lg_ref[...]
    m = jnp.max(logits, axis=-1, keepdims=True)
    e = jnp.exp(logits - m)
    probs = e / jnp.sum(e, axis=-1, keepdims=True)
    gp_ref[...] = probs

    lane = jax.lax.broadcasted_iota(jnp.int32, probs.shape, 1)
    desc = (63 - lane).astype(jnp.float32)
    p = probs
    vals = []
    scs = []
    for _ in range(_K):
        mx = jnp.max(p, axis=-1, keepdims=True)
        score = jnp.where(p >= mx, desc, -1.0)
        sc = jnp.max(score, axis=-1, keepdims=True)
        vals.append(mx)
        scs.append(sc)
        p = jnp.where(score == sc, -1.0, p)
    tv = jnp.concatenate(vals, axis=1)
    tp_ref[...] = tv / jnp.sum(tv, axis=1, keepdims=True)
    ti_ref[...] = (63.0 - jnp.concatenate(scs, axis=1)).astype(jnp.int32)

    x = x_ref[...]
    h0 = jnp.dot(x, wi_ref[...], preferred_element_type=jnp.float32)
    h0 = jnp.maximum(h0 + bi_ref[...], 0.0)

    mu = jnp.mean(h0, axis=-1, keepdims=True)
    var = jnp.mean((h0 - mu) ** 2, axis=-1, keepdims=True)
    t = (h0 - mu) * jax.lax.rsqrt(var + 1e-5) * g1_ref[...] + be1_ref[...]
    t = jnp.maximum(t, 0.0)

    h1 = jnp.dot(t, wh1_ref[...], preferred_element_type=jnp.float32)
    h1 = h1 + bh1_ref[...] + h0

    mu2 = jnp.mean(h1, axis=-1, keepdims=True)
    var2 = jnp.mean((h1 - mu2) ** 2, axis=-1, keepdims=True)
    t2 = (h1 - mu2) * jax.lax.rsqrt(var2 + 1e-5) * g2_ref[...] + be2_ref[...]
    t2 = jnp.maximum(t2, 0.0)

    h2 = jnp.dot(t2, wh2_ref[...], preferred_element_type=jnp.float32)
    h2 = h2 + bh2_ref[...]

    new_logits = jnp.dot(h2, wo_ref[...], preferred_element_type=jnp.float32)
    lg_ref[...] = new_logits + bo_ref[...]


@functools.partial(jax.jit, static_argnames=("interpret",))
def _run(x, wi, bi, g1, be1, wh1, bh1, g2, be2, wh2, bh2, wo, bo,
         interpret=False):
    n = x.shape[0]
    d = x.shape[1]
    nblk = n // _TILE
    grid = (nblk + 1,)

    def xmap(i):
        return (jnp.minimum(i, nblk - 1), 0)

    def omap(i):
        return (jnp.maximum(i - 1, 0), 0)

    def rep(i):
        return (0, 0)

    full = lambda s: pl.BlockSpec(s, rep)
    out_shapes = (
        jax.ShapeDtypeStruct((n, _K), jnp.float32),
        jax.ShapeDtypeStruct((n, _K), jnp.int32),
        jax.ShapeDtypeStruct((n, _N_EXPERTS), jnp.float32),
    )
    return pl.pallas_call(
        _body,
        grid=grid,
        in_specs=[
            pl.BlockSpec((_TILE, d), xmap),
            full((d, 256)), full((1, 256)), full((1, 256)), full((1, 256)),
            full((256, 256)), full((1, 256)), full((1, 256)), full((1, 256)),
            full((256, 128)), full((1, 128)),
            full((128, _N_EXPERTS)), full((1, _N_EXPERTS)),
        ],
        out_specs=(
            pl.BlockSpec((_TILE, _K), omap),
            pl.BlockSpec((_TILE, _K), omap),
            pl.BlockSpec((_TILE, _N_EXPERTS), omap),
        ),
        out_shape=out_shapes,
        scratch_shapes=[pltpu.VMEM((_TILE, _N_EXPERTS), jnp.float32)],
        compiler_params=pltpu.CompilerParams(
            dimension_semantics=("arbitrary",),
        ),
        interpret=interpret,
    )(x, wi, bi, g1, be1, wh1, bh1, g2, be2, wh2, bh2, wo, bo)


def kernel(x, W_in, b_in, ln1_g, ln1_b, W_h1, b_h1, ln2_g, ln2_b,
           W_h2, b_h2, W_out, b_out, temperature):
    temp = jnp.clip(temperature, 0.5, 5.0)[0]
    wo = (W_out.T / temp).astype(jnp.float32)
    bo = (b_out / temp).reshape(1, -1)
    r = lambda v: v.reshape(1, -1)
    return _run(x, W_in.T, r(b_in), r(ln1_g), r(ln1_b), W_h1.T, r(b_h1),
                r(ln2_g), r(ln2_b), W_h2.T, r(b_h2), wo, bo)


# speedup vs baseline: 1.3011x; 1.0116x over previous
import functools

import jax
import jax.numpy as jnp
from jax.experimental import pallas as pl
from jax.experimental.pallas import tpu as pltpu

_TILE = 512
_N_EXPERTS = 64
_K = 8


def _body(x_ref, wi_ref, bi_ref, g1_ref, be1_ref, wh1_ref, bh1_ref,
          g2_ref, be2_ref, wh2_ref, bh2_ref, wo_ref, bo_ref,
          tp_ref, ti_ref, gp_ref, lg_ref):
    logits = lg_ref[...]
    m = jnp.max(logits, axis=-1, keepdims=True)
    e = jnp.exp(logits - m)
    probs = e / jnp.sum(e, axis=-1, keepdims=True)
    gp_ref[...] = probs

    lane = jax.lax.broadcasted_iota(jnp.int32, probs.shape, 1)
    desc = (63 - lane).astype(jnp.float32)
    p = probs
    vals = []
    scs = []
    for _ in range(_K):
        mx = jnp.max(p, axis=-1, keepdims=True)
        score = jnp.where(p >= mx, desc, -1.0)
        sc = jnp.max(score, axis=-1, keepdims=True)
        vals.append(mx)
        scs.append(sc)
        p = jnp.where(score == sc, -1.0, p)
    tv = jnp.concatenate(vals, axis=1)
    tp_ref[...] = tv / jnp.sum(tv, axis=1, keepdims=True)
    ti_ref[...] = (63.0 - jnp.concatenate(scs, axis=1)).astype(jnp.int32)

    x = x_ref[...].astype(jnp.bfloat16)
    h0 = jnp.dot(x, wi_ref[...], preferred_element_type=jnp.float32)
    h0 = jnp.maximum(h0 + bi_ref[...], 0.0)

    mu = jnp.mean(h0, axis=-1, keepdims=True)
    var = jnp.mean((h0 - mu) ** 2, axis=-1, keepdims=True)
    t = (h0 - mu) / jnp.sqrt(var + 1e-5) * g1_ref[...] + be1_ref[...]
    t = jnp.maximum(t, 0.0).astype(jnp.bfloat16)

    h1 = jnp.dot(t, wh1_ref[...], preferred_element_type=jnp.float32)
    h1 = h1 + bh1_ref[...] + h0

    mu2 = jnp.mean(h1, axis=-1, keepdims=True)
    var2 = jnp.mean((h1 - mu2) ** 2, axis=-1, keepdims=True)
    t2 = (h1 - mu2) / jnp.sqrt(var2 + 1e-5) * g2_ref[...] + be2_ref[...]
    t2 = jnp.maximum(t2, 0.0).astype(jnp.bfloat16)

    h2 = jnp.dot(t2, wh2_ref[...], preferred_element_type=jnp.float32)
    h2 = (h2 + bh2_ref[...]).astype(jnp.bfloat16)

    new_logits = jnp.dot(h2, wo_ref[...], preferred_element_type=jnp.float32)
    lg_ref[...] = new_logits + bo_ref[...]


@functools.partial(jax.jit, static_argnames=("interpret",))
def _run(x, wi, bi, g1, be1, wh1, bh1, g2, be2, wh2, bh2, wo, bo,
         interpret=False):
    n = x.shape[0]
    d = x.shape[1]
    nblk = n // _TILE
    grid = (nblk + 1,)

    def xmap(i):
        return (jnp.minimum(i, nblk - 1), 0)

    def omap(i):
        return (jnp.maximum(i - 1, 0), 0)

    def rep(i):
        return (0, 0)

    full = lambda s: pl.BlockSpec(s, rep)
    out_shapes = (
        jax.ShapeDtypeStruct((n, _K), jnp.float32),
        jax.ShapeDtypeStruct((n, _K), jnp.int32),
        jax.ShapeDtypeStruct((n, _N_EXPERTS), jnp.float32),
    )
    return pl.pallas_call(
        _body,
        grid=grid,
        in_specs=[
            pl.BlockSpec((_TILE, d), xmap),
            full((d, 256)), full((1, 256)), full((1, 256)), full((1, 256)),
            full((256, 256)), full((1, 256)), full((1, 256)), full((1, 256)),
            full((256, 128)), full((1, 128)),
            full((128, _N_EXPERTS)), full((1, _N_EXPERTS)),
        ],
        out_specs=(
            pl.BlockSpec((_TILE, _K), omap),
            pl.BlockSpec((_TILE, _K), omap),
            pl.BlockSpec((_TILE, _N_EXPERTS), omap),
        ),
        out_shape=out_shapes,
        scratch_shapes=[pltpu.VMEM((_TILE, _N_EXPERTS), jnp.float32)],
        compiler_params=pltpu.CompilerParams(
            dimension_semantics=("arbitrary",),
        ),
        interpret=interpret,
    )(x, wi, bi, g1, be1, wh1, bh1, g2, be2, wh2, bh2, wo, bo)


def kernel(x, W_in, b_in, ln1_g, ln1_b, W_h1, b_h1, ln2_g, ln2_b,
           W_h2, b_h2, W_out, b_out, temperature):
    temp = jnp.clip(temperature, 0.5, 5.0)[0]
    wo = (W_out.T / temp).astype(jnp.bfloat16)
    bo = (b_out / temp).reshape(1, -1)
    r = lambda v: v.reshape(1, -1)
    b16 = lambda w: w.astype(jnp.bfloat16)
    return _run(x, b16(W_in.T), r(b_in), r(ln1_g), r(ln1_b), b16(W_h1.T),
                r(b_h1), r(ln2_g), r(ln2_b), b16(W_h2.T), r(b_h2), wo, bo)
